# baseline (speedup 1.0000x reference)
import jax
import jax.numpy as jnp
from jax import lax
from jax.experimental import pallas as pl
from jax.experimental.pallas import tpu as pltpu
from jax.experimental.pallas import tpu_sc as plsc

_NC = 2
_NS = 16
_NW = _NC * _NS
_CHUNK = 64


def _sc_body(table_hbm, out_hbm, buf):
    B, S, _ = out_hbm.shape
    rows_per_w = S // _NW
    wid = lax.axis_index("s") * _NC + lax.axis_index("c")
    base = wid * rows_per_w
    for c in range(rows_per_w // _CHUNK):
        r = base + c * _CHUNK
        pltpu.sync_copy(table_hbm.at[pl.ds(r, _CHUNK)], buf)
        for b in range(B):
            pltpu.sync_copy(buf, out_hbm.at[b, pl.ds(r, _CHUNK)])


def kernel(x, table):
    B, S, D = x.shape
    f = pl.kernel(
        _sc_body,
        out_type=jax.ShapeDtypeStruct((B, S, D), table.dtype),
        mesh=plsc.VectorSubcoreMesh(core_axis_name="c", subcore_axis_name="s"),
        scratch_types=[pltpu.VMEM((_CHUNK, D), table.dtype)],
    )
    return f(table)
